# Initial kernel scaffold; baseline (speedup 1.0000x reference)
import jax
import jax.numpy as jnp
from jax.experimental import pallas as pl


def kernel(x, Wr, br, W1, W3, W2):
    raise NotImplementedError("write your pallas kernel here")


# speedup vs baseline: 1.4379x; 1.4379x over previous
import functools

import jax
import jax.numpy as jnp
from jax.experimental import pallas as pl
from jax.experimental.pallas import tpu as pltpu

B, S, D, H, E, TOPK = 2, 2048, 768, 1024, 8, 2
T = B * S
TT = 2048


def _moe_body(xf_ref, Wr_ref, br_ref, W1_ref, W3_ref, W2_ref, out_ref):
    e = pl.program_id(1)
    xf = xf_ref[...]
    logits = jnp.dot(xf, Wr_ref[...], preferred_element_type=jnp.float32)
    logits = logits + br_ref[...]
    probs = jax.nn.softmax(logits, axis=-1)
    cols = jax.lax.broadcasted_iota(jnp.int32, probs.shape, 1)
    a1 = jnp.argmax(probs, axis=-1)
    p1 = jnp.max(probs, axis=-1)
    masked = jnp.where(cols == a1[:, None], -jnp.inf, probs)
    a2 = jnp.argmax(masked, axis=-1)
    p2 = jnp.max(masked, axis=-1)
    denom = p1 + p2
    we = (p1 * (a1 == e) + p2 * (a2 == e)) / denom

    w1 = W1_ref[0]
    w3 = W3_ref[0]
    w2 = W2_ref[0]
    h1 = jnp.dot(xf, w1, preferred_element_type=jnp.float32)
    h3 = jnp.dot(xf, w3, preferred_element_type=jnp.float32)
    y = jnp.dot(jax.nn.silu(h1) * h3, w2, preferred_element_type=jnp.float32)
    contrib = we[:, None] * y

    @pl.when(e == 0)
    def _():
        out_ref[...] = contrib

    @pl.when(e != 0)
    def _():
        out_ref[...] += contrib


@jax.jit
def _moe(xf, Wr, br2, W1, W3, W2):
    n_tt = T // TT
    return pl.pallas_call(
        _moe_body,
        grid=(n_tt, E),
        in_specs=[
            pl.BlockSpec((TT, D), lambda t, e: (t, 0)),
            pl.BlockSpec((D, E), lambda t, e: (0, 0)),
            pl.BlockSpec((1, E), lambda t, e: (0, 0)),
            pl.BlockSpec((1, D, H), lambda t, e: (e, 0, 0)),
            pl.BlockSpec((1, D, H), lambda t, e: (e, 0, 0)),
            pl.BlockSpec((1, H, D), lambda t, e: (e, 0, 0)),
        ],
        out_specs=pl.BlockSpec((TT, D), lambda t, e: (t, 0)),
        out_shape=jax.ShapeDtypeStruct((T, D), jnp.float32),
        compiler_params=pltpu.CompilerParams(
            dimension_semantics=("arbitrary", "arbitrary"),
        ),
    )(xf, Wr, br2, W1, W3, W2)


def kernel(x, Wr, br, W1, W3, W2):
    b, s, d = x.shape
    xf = x.reshape(b * s, d)
    out = _moe(xf, Wr, br.reshape(1, E), W1, W3, W2)
    return out.reshape(b, s, d)
